# Initial kernel scaffold; baseline (speedup 1.0000x reference)
import jax
import jax.numpy as jnp
from jax.experimental import pallas as pl


def kernel(feat, edge_index1, edge_index2, W1, a_l1, a_r1, b1, W2, a_l2, a_r2, b2):
    raise NotImplementedError("write your pallas kernel here")


# speedup vs baseline: 1.1556x; 1.1556x over previous
import functools

import jax
import jax.numpy as jnp
from jax.experimental import pallas as pl
from jax.experimental.pallas import tpu as pltpu

N = 10000
E = 320000
D_IN = 128
H1 = 4
D1 = 128
D2 = 256

ROWS = 1000


def _mm_body(x_ref, w_ref, h_ref):
    h_ref[...] = jnp.dot(x_ref[...], w_ref[...], preferred_element_type=jnp.float32)


def _mm(x, W):
    grid = N // ROWS
    K = x.shape[1]
    M = W.shape[1]
    return pl.pallas_call(
        _mm_body,
        grid=(grid,),
        in_specs=[
            pl.BlockSpec((ROWS, K), lambda i: (i, 0)),
            pl.BlockSpec((K, M), lambda i: (0, 0)),
        ],
        out_specs=pl.BlockSpec((ROWS, M), lambda i: (i, 0)),
        out_shape=jax.ShapeDtypeStruct((N, M), jnp.float32),
    )(x, W)


def _edge_agg(h, el, er, src, dst, num_heads, d_head):
    e = el[src] + er[dst]
    e = jnp.where(e >= 0, e, 0.2 * e)
    ee = jnp.exp(e)
    denom = jax.ops.segment_sum(ee, dst, num_segments=N)
    hh = h.reshape(N, num_heads, d_head)
    raw = jax.ops.segment_sum(hh[src] * ee[:, :, None], dst, num_segments=N)
    return raw / (denom[:, :, None] + 1e-9)


def kernel(feat, edge_index1, edge_index2, W1, a_l1, a_r1, b1, W2, a_l2, a_r2, b2):
    src1, dst1 = edge_index1[0], edge_index1[1]
    src2, dst2 = edge_index2[0], edge_index2[1]

    h1 = _mm(feat, W1)
    hh1 = h1.reshape(N, H1, D1)
    el1 = jnp.sum(hh1 * a_l1[None], axis=-1)
    er1 = jnp.sum(hh1 * a_r1[None], axis=-1)
    agg1 = _edge_agg(h1, el1, er1, src1, dst1, H1, D1)
    x1 = agg1.reshape(N, H1 * D1) + b1
    x1 = jax.nn.elu(x1)

    h2 = _mm(x1, W2)
    el2 = jnp.sum(h2 * a_l2, axis=-1, keepdims=True)
    er2 = jnp.sum(h2 * a_r2, axis=-1, keepdims=True)
    agg2 = _edge_agg(h2, el2, er2, src2, dst2, 1, D2)
    z = agg2.reshape(N, D2) + b2
    return z


# speedup vs baseline: 19.1542x; 16.5744x over previous
import functools

import jax
import jax.numpy as jnp
from jax import lax
from jax.experimental import pallas as pl
from jax.experimental.pallas import tpu as pltpu
from jax.experimental.pallas import tpu_sc as plsc

N = 10000
E = 320000
D_IN = 128
H1 = 4
D1 = 128
D2 = 256

ROWS = 1000
K = 80
N_TILES = 16
E_PER_TILE = E // N_TILES
RPT = 624
ZROWS = 104


def _wa_body(w_ref, a_ref, o_ref):
    o_ref[...] = jnp.dot(w_ref[...], a_ref[...], preferred_element_type=jnp.float32)


def _wa(W, A):
    return pl.pallas_call(
        _wa_body,
        out_shape=jax.ShapeDtypeStruct((W.shape[0], A.shape[1]), jnp.float32),
    )(W, A)


def _proj1_body(x_ref, w_ref, wa_ref, hcat_ref, lr_ref):
    h = jnp.dot(x_ref[...], w_ref[...], preferred_element_type=jnp.float32)
    for g in range(H1):
        hcat_ref[g] = h[:, g * 128:(g + 1) * 128]
    lr_ref[...] = jnp.dot(x_ref[...], wa_ref[...], preferred_element_type=jnp.float32)


def _proj1(x, W, WA):
    return pl.pallas_call(
        _proj1_body,
        grid=(N // ROWS,),
        in_specs=[
            pl.BlockSpec((ROWS, D_IN), lambda i: (i, 0)),
            pl.BlockSpec((D_IN, H1 * D1), lambda i: (0, 0)),
            pl.BlockSpec((D_IN, 128), lambda i: (0, 0)),
        ],
        out_specs=[
            pl.BlockSpec((H1, ROWS, 128), lambda i: (0, i, 0)),
            pl.BlockSpec((ROWS, 128), lambda i: (i, 0)),
        ],
        out_shape=[
            jax.ShapeDtypeStruct((H1, N, 128), jnp.float32),
            jax.ShapeDtypeStruct((N, 128), jnp.float32),
        ],
    )(x, W, WA)


def _proj2_body(raw_ref, den_ref, b_ref, w_ref, wa_ref, hcat_ref, lr_ref):
    parts = []
    for g in range(H1):
        den = den_ref[:, g:g + 1] + 1e-9
        parts.append(raw_ref[g] / den)
    x1 = jnp.concatenate(parts, axis=1) + b_ref[...]
    x1 = jnp.where(x1 > 0, x1, jnp.exp(jnp.minimum(x1, 0.0)) - 1.0)
    h2 = jnp.dot(x1, w_ref[...], preferred_element_type=jnp.float32)
    hcat_ref[0] = h2[:, :128]
    hcat_ref[1] = h2[:, 128:]
    lr_ref[...] = jnp.dot(x1, wa_ref[...], preferred_element_type=jnp.float32)


def _proj2(raw1, den1, b1, W2, WA2):
    return pl.pallas_call(
        _proj2_body,
        grid=(N // ROWS,),
        in_specs=[
            pl.BlockSpec((H1, ROWS, 128), lambda i: (0, i, 0)),
            pl.BlockSpec((ROWS, H1), lambda i: (i, 0)),
            pl.BlockSpec((1, H1 * D1), lambda i: (0, 0)),
            pl.BlockSpec((H1 * D1, D2), lambda i: (0, 0)),
            pl.BlockSpec((H1 * D1, 128), lambda i: (0, 0)),
        ],
        out_specs=[
            pl.BlockSpec((2, ROWS, 128), lambda i: (0, i, 0)),
            pl.BlockSpec((ROWS, 128), lambda i: (i, 0)),
        ],
        out_shape=[
            jax.ShapeDtypeStruct((2, N, 128), jnp.float32),
            jax.ShapeDtypeStruct((N, 128), jnp.float32),
        ],
    )(raw1, den1, b1, W2, WA2)


def _final_body(raw_ref, den_ref, b_ref, z_ref):
    parts = []
    for g in range(2):
        den = den_ref[:, g:g + 1] + 1e-9
        parts.append(raw_ref[g] / den)
    z_ref[...] = jnp.concatenate(parts, axis=1) + b_ref[...]


def _final(raw2, den2, b2):
    return pl.pallas_call(
        _final_body,
        grid=(N // ROWS,),
        in_specs=[
            pl.BlockSpec((2, ROWS, 128), lambda i: (0, i, 0)),
            pl.BlockSpec((ROWS, 2), lambda i: (i, 0)),
            pl.BlockSpec((1, D2), lambda i: (0, 0)),
        ],
        out_specs=pl.BlockSpec((ROWS, D2), lambda i: (i, 0)),
        out_shape=jax.ShapeDtypeStruct((N, D2), jnp.float32),
    )(raw2, den2, b2)


def _make_sc_edge(G, el_stride):
    g_per_core = G // 2
    n_chunks = E_PER_TILE // K
    mesh = plsc.VectorSubcoreMesh(core_axis_name="c", subcore_axis_name="s")

    @functools.partial(
        pl.kernel,
        mesh=mesh,
        out_type=[
            jax.ShapeDtypeStruct((G * N, 128), jnp.float32),
            jax.ShapeDtypeStruct((G * N,), jnp.float32),
        ],
        scratch_types=[
            pltpu.VMEM((RPT,), jnp.float32),
            pltpu.VMEM((RPT,), jnp.float32),
            pltpu.VMEM((K,), jnp.int32),
            pltpu.VMEM((K,), jnp.int32),
            pltpu.VMEM((K,), jnp.int32),
            pltpu.VMEM((K,), jnp.int32),
            pltpu.VMEM((K,), jnp.int32),
            pltpu.VMEM((K,), jnp.float32),
            pltpu.VMEM((K,), jnp.float32),
            pltpu.VMEM((K + 16,), jnp.float32),
            pltpu.VMEM((K, 128), jnp.float32),
            pltpu.VMEM((K, 128), jnp.float32),
            pltpu.VMEM((ZROWS, 128), jnp.float32),
            pltpu.VMEM_SHARED((N, 128), jnp.float32),
            pltpu.VMEM_SHARED((N,), jnp.float32),
            pltpu.SemaphoreType.DMA,
            pltpu.SemaphoreType.DMA,
            pltpu.SemaphoreType.DMA,
        ],
    )
    def sc_edge(src_hbm, dst_hbm, hcat_hbm, el_hbm, er_hbm, out_hbm, den_hbm,
                zbuf1, den_stage, src_v, dst_v, gidx_v, eli_v, eri_v, els_v, ers_v,
                ee_v, rows_v, out_v, zbuf, acc, den_sp, sem, sem2, sem3):
        c = lax.axis_index("c")
        s = lax.axis_index("s")
        zero16 = jnp.zeros((16,), jnp.float32)

        def zero_zbuf(r, _):
            for j in range(8):
                zbuf[r, pl.ds(j * 16, 16)] = zero16
            return 0

        lax.fori_loop(0, ZROWS, zero_zbuf, 0)

        def zero_zbuf1(i, _):
            zbuf1[pl.ds(i * 16, 16)] = zero16
            return 0

        lax.fori_loop(0, RPT // 16, zero_zbuf1, 0)

        base = s * E_PER_TILE
        row0 = s * RPT

        for kg in range(g_per_core):
            g = c * g_per_core + kg
            pltpu.sync_copy(zbuf1, den_sp.at[pl.ds(s * RPT, RPT)])
            for z in range(RPT // ZROWS):
                pltpu.sync_copy(zbuf, acc.at[pl.ds(row0 + z * ZROWS, ZROWS)])

            @pl.when(s == N_TILES - 1)
            def _():
                pltpu.sync_copy(zbuf.at[pl.ds(0, 16)], acc.at[pl.ds(N - 16, 16)])
                pltpu.sync_copy(zbuf1.at[pl.ds(0, 16)], den_sp.at[pl.ds(N - 16, 16)])

            plsc.subcore_barrier()

            def chunk_body(i, _):
                e0 = base + i * K
                pltpu.sync_copy(src_hbm.at[pl.ds(e0, K)], src_v)
                pltpu.sync_copy(dst_hbm.at[pl.ds(e0, K)], dst_v)
                for q in range(K // 16):
                    sl = pl.ds(q * 16, 16)
                    s16 = src_v[sl]
                    d16 = dst_v[sl]
                    gidx_v[sl] = s16 + g * N
                    eli_v[sl] = s16 + g * el_stride
                    eri_v[sl] = d16 + g * el_stride
                cp1 = pltpu.async_copy(el_hbm.at[eli_v], els_v, sem)
                cp2 = pltpu.async_copy(er_hbm.at[eri_v], ers_v, sem2)
                cp3 = pltpu.async_copy(hcat_hbm.at[gidx_v], rows_v, sem3)
                cp1.wait()
                cp2.wait()
                for q in range(K // 16):
                    sl = pl.ds(q * 16, 16)
                    sc = els_v[sl] + ers_v[sl]
                    sc = jnp.where(sc >= 0, sc, 0.2 * sc)
                    ee_v[sl] = jnp.exp(sc)
                cp3.wait()

                def row_body(r, _):
                    bc = jnp.zeros((16,), jnp.float32) + ee_v[pl.ds(r, 16)][0]
                    for j in range(8):
                        out_v[r, pl.ds(j * 16, 16)] = rows_v[r, pl.ds(j * 16, 16)] * bc
                    return 0

                lax.fori_loop(0, K, row_body, 0)
                pltpu.sync_copy(out_v, acc.at[dst_v], add=True)
                pltpu.sync_copy(ee_v.at[pl.ds(0, K)], den_sp.at[dst_v], add=True)
                return 0

            lax.fori_loop(0, n_chunks, chunk_body, 0)
            plsc.subcore_barrier()
            pltpu.sync_copy(acc.at[pl.ds(row0, RPT)],
                            out_hbm.at[pl.ds(g * N + row0, RPT)])

            pltpu.sync_copy(den_sp.at[pl.ds(s * RPT, RPT)], den_stage)
            pltpu.sync_copy(den_stage, den_hbm.at[pl.ds(g * N + s * RPT, RPT)])

            @pl.when(s == N_TILES - 1)
            def _():
                pltpu.sync_copy(acc.at[pl.ds(N - 16, 16)],
                                out_hbm.at[pl.ds(g * N + N - 16, 16)])
                pltpu.sync_copy(den_sp.at[pl.ds(N - 16, 16)],
                                den_stage.at[pl.ds(0, 16)])
                pltpu.sync_copy(den_stage.at[pl.ds(0, 16)],
                                den_hbm.at[pl.ds(g * N + N - 16, 16)])

    return sc_edge


_sc_edge1 = _make_sc_edge(H1, N)
_sc_edge2 = _make_sc_edge(2, 0)


def kernel(feat, edge_index1, edge_index2, W1, a_l1, a_r1, b1, W2, a_l2, a_r2, b2):
    src1, dst1 = edge_index1[0], edge_index1[1]
    src2, dst2 = edge_index2[0], edge_index2[1]

    row1 = jnp.arange(H1 * D1)
    col = jnp.arange(128)
    alf = a_l1.reshape(-1)
    arf = a_r1.reshape(-1)
    A1 = (jnp.where(col[None, :] == (row1 // D1)[:, None], alf[:, None], 0.0)
          + jnp.where(col[None, :] == H1 + (row1 // D1)[:, None], arf[:, None], 0.0))
    A2 = (jnp.where(col[None, :] == 0, a_l2[0][:, None], 0.0)
          + jnp.where(col[None, :] == 1, a_r2[0][:, None], 0.0))
    WA1 = _wa(W1, A1)
    WA2 = _wa(W2, A2)

    hcat1, lr1 = _proj1(feat, W1, WA1)
    el1 = lr1[:, :H1].T.reshape(-1)
    er1 = lr1[:, H1:2 * H1].T.reshape(-1)
    raw1, den1 = _sc_edge1(src1, dst1, hcat1.reshape(H1 * N, 128), el1, er1)
    raw1 = raw1.reshape(H1, N, 128)
    den1 = den1.reshape(H1, N).T

    hcat2, lr2 = _proj2(raw1, den1, b1.reshape(1, -1), W2, WA2)
    el2 = lr2[:, 0]
    er2 = lr2[:, 1]
    raw2, den2 = _sc_edge2(src2, dst2, hcat2.reshape(2 * N, 128), el2, er2)
    raw2 = raw2.reshape(2, N, 128)
    den2 = den2.reshape(2, N).T

    return _final(raw2, den2, b2.reshape(1, -1))


# speedup vs baseline: 32.3026x; 1.6865x over previous
import functools

import jax
import jax.numpy as jnp
from jax import lax
from jax.experimental import pallas as pl
from jax.experimental.pallas import tpu as pltpu
from jax.experimental.pallas import tpu_sc as plsc

N = 10000
E = 320000
D_IN = 128
H1 = 4
D1 = 128
D2 = 256

ROWS = 1000
K = 80
N_TILES = 16
E_PER_TILE = E // N_TILES
RPT = 624
ZROWS = 104


def _wa_body(w_ref, a_ref, o_ref):
    o_ref[...] = jnp.dot(w_ref[...], a_ref[...], preferred_element_type=jnp.float32)


def _wa(W, A):
    return pl.pallas_call(
        _wa_body,
        out_shape=jax.ShapeDtypeStruct((W.shape[0], A.shape[1]), jnp.float32),
    )(W, A)


def _proj1_body(x_ref, w_ref, wa_ref, hcat_ref, lr_ref):
    h = jnp.dot(x_ref[...], w_ref[...], preferred_element_type=jnp.float32)
    for g in range(H1):
        hcat_ref[g] = h[:, g * 128:(g + 1) * 128]
    lr_ref[...] = jnp.dot(x_ref[...], wa_ref[...], preferred_element_type=jnp.float32)


def _proj1(x, W, WA):
    return pl.pallas_call(
        _proj1_body,
        grid=(N // ROWS,),
        in_specs=[
            pl.BlockSpec((ROWS, D_IN), lambda i: (i, 0)),
            pl.BlockSpec((D_IN, H1 * D1), lambda i: (0, 0)),
            pl.BlockSpec((D_IN, 128), lambda i: (0, 0)),
        ],
        out_specs=[
            pl.BlockSpec((H1, ROWS, 128), lambda i: (0, i, 0)),
            pl.BlockSpec((ROWS, 128), lambda i: (i, 0)),
        ],
        out_shape=[
            jax.ShapeDtypeStruct((H1, N, 128), jnp.float32),
            jax.ShapeDtypeStruct((N, 128), jnp.float32),
        ],
    )(x, W, WA)


def _proj2_body(raw_ref, den_ref, b_ref, w_ref, wa_ref, hcat_ref, lr_ref):
    parts = []
    for g in range(H1):
        den = den_ref[:, g:g + 1] + 1e-9
        parts.append(raw_ref[g] / den)
    x1 = jnp.concatenate(parts, axis=1) + b_ref[...]
    x1 = jnp.where(x1 > 0, x1, jnp.exp(jnp.minimum(x1, 0.0)) - 1.0)
    h2 = jnp.dot(x1, w_ref[...], preferred_element_type=jnp.float32)
    hcat_ref[0] = h2[:, :128]
    hcat_ref[1] = h2[:, 128:]
    lr_ref[...] = jnp.dot(x1, wa_ref[...], preferred_element_type=jnp.float32)


def _proj2(raw1, den1, b1, W2, WA2):
    return pl.pallas_call(
        _proj2_body,
        grid=(N // ROWS,),
        in_specs=[
            pl.BlockSpec((H1, ROWS, 128), lambda i: (0, i, 0)),
            pl.BlockSpec((ROWS, H1), lambda i: (i, 0)),
            pl.BlockSpec((1, H1 * D1), lambda i: (0, 0)),
            pl.BlockSpec((H1 * D1, D2), lambda i: (0, 0)),
            pl.BlockSpec((H1 * D1, 128), lambda i: (0, 0)),
        ],
        out_specs=[
            pl.BlockSpec((2, ROWS, 128), lambda i: (0, i, 0)),
            pl.BlockSpec((ROWS, 128), lambda i: (i, 0)),
        ],
        out_shape=[
            jax.ShapeDtypeStruct((2, N, 128), jnp.float32),
            jax.ShapeDtypeStruct((N, 128), jnp.float32),
        ],
    )(raw1, den1, b1, W2, WA2)


def _final_body(raw_ref, den_ref, b_ref, z_ref):
    parts = []
    for g in range(2):
        den = den_ref[:, g:g + 1] + 1e-9
        parts.append(raw_ref[g] / den)
    z_ref[...] = jnp.concatenate(parts, axis=1) + b_ref[...]


def _final(raw2, den2, b2):
    return pl.pallas_call(
        _final_body,
        grid=(N // ROWS,),
        in_specs=[
            pl.BlockSpec((2, ROWS, 128), lambda i: (0, i, 0)),
            pl.BlockSpec((ROWS, 2), lambda i: (i, 0)),
            pl.BlockSpec((1, D2), lambda i: (0, 0)),
        ],
        out_specs=pl.BlockSpec((ROWS, D2), lambda i: (i, 0)),
        out_shape=jax.ShapeDtypeStruct((N, D2), jnp.float32),
    )(raw2, den2, b2)


def _make_sc_edge(G, el_stride):
    g_per_core = G // 2
    n_chunks = E_PER_TILE // K
    mesh = plsc.VectorSubcoreMesh(core_axis_name="c", subcore_axis_name="s")

    @functools.partial(
        pl.kernel,
        mesh=mesh,
        out_type=[
            jax.ShapeDtypeStruct((G * N, 128), jnp.float32),
            jax.ShapeDtypeStruct((G * N,), jnp.float32),
        ],
        scratch_types=[
            pltpu.VMEM((RPT,), jnp.float32),
            pltpu.VMEM((RPT,), jnp.float32),
            pltpu.VMEM((K + 16,), jnp.float32),
            pltpu.VMEM((K, 128), jnp.float32),
            pltpu.VMEM((ZROWS, 128), jnp.float32),
            pltpu.VMEM_SHARED((N, 128), jnp.float32),
            pltpu.VMEM_SHARED((N,), jnp.float32),
        ] + 2 * [
            pltpu.VMEM((K,), jnp.int32),
            pltpu.VMEM((K,), jnp.int32),
            pltpu.VMEM((K,), jnp.int32),
            pltpu.VMEM((K,), jnp.int32),
            pltpu.VMEM((K,), jnp.int32),
            pltpu.VMEM((K,), jnp.float32),
            pltpu.VMEM((K,), jnp.float32),
            pltpu.VMEM((K, 128), jnp.float32),
            pltpu.SemaphoreType.DMA,
            pltpu.SemaphoreType.DMA,
            pltpu.SemaphoreType.DMA,
        ],
    )
    def sc_edge(src_hbm, dst_hbm, hcat_hbm, el_hbm, er_hbm, out_hbm, den_hbm,
                zbuf1, den_stage, ee_v, out_v, zbuf, acc, den_sp, *bufflat):
        bufs = (bufflat[:11], bufflat[11:])
        c = lax.axis_index("c")
        s = lax.axis_index("s")
        zero16 = jnp.zeros((16,), jnp.float32)

        def zero_zbuf(r, _):
            for j in range(8):
                zbuf[r, pl.ds(j * 16, 16)] = zero16
            return 0

        lax.fori_loop(0, ZROWS, zero_zbuf, 0)

        def zero_zbuf1(i, _):
            zbuf1[pl.ds(i * 16, 16)] = zero16
            return 0

        lax.fori_loop(0, RPT // 16, zero_zbuf1, 0)

        base = s * E_PER_TILE
        row0 = s * RPT

        for kg in range(g_per_core):
            g = c * g_per_core + kg
            pltpu.sync_copy(zbuf1, den_sp.at[pl.ds(s * RPT, RPT)])
            for z in range(RPT // ZROWS):
                pltpu.sync_copy(zbuf, acc.at[pl.ds(row0 + z * ZROWS, ZROWS)])

            @pl.when(s == N_TILES - 1)
            def _():
                pltpu.sync_copy(zbuf.at[pl.ds(0, 16)], acc.at[pl.ds(N - 16, 16)])
                pltpu.sync_copy(zbuf1.at[pl.ds(0, 16)], den_sp.at[pl.ds(N - 16, 16)])

            plsc.subcore_barrier()

            def fire_sd(B, i):
                e0 = base + i * K
                pltpu.async_copy(src_hbm.at[pl.ds(e0, K)], B[0], B[8])
                pltpu.async_copy(dst_hbm.at[pl.ds(e0, K)], B[1], B[8])

            def wait_sd(B):
                pltpu.make_async_copy(src_hbm.at[pl.ds(0, K)], B[0], B[8]).wait()
                pltpu.make_async_copy(dst_hbm.at[pl.ds(0, K)], B[1], B[8]).wait()

            def prep_fire_gathers(B):
                for q in range(K // 16):
                    sl = pl.ds(q * 16, 16)
                    s16 = B[0][sl]
                    d16 = B[1][sl]
                    B[2][sl] = s16 + g * N
                    B[3][sl] = s16 + g * el_stride
                    B[4][sl] = d16 + g * el_stride
                pltpu.async_copy(el_hbm.at[B[3]], B[5], B[9])
                pltpu.async_copy(er_hbm.at[B[4]], B[6], B[9])
                pltpu.async_copy(hcat_hbm.at[B[2]], B[7], B[10])

            def wait_gathers(B):
                pltpu.make_async_copy(el_hbm.at[B[3]], B[5], B[9]).wait()
                pltpu.make_async_copy(er_hbm.at[B[4]], B[6], B[9]).wait()
                pltpu.make_async_copy(hcat_hbm.at[B[2]], B[7], B[10]).wait()

            def consume(B):
                for q in range(K // 16):
                    sl = pl.ds(q * 16, 16)
                    sc = B[5][sl] + B[6][sl]
                    sc = jnp.where(sc >= 0, sc, 0.2 * sc)
                    ee_v[sl] = jnp.exp(sc)

                def row_body(r2, _):
                    for u in range(2):
                        r = r2 * 2 + u
                        bc = jnp.zeros((16,), jnp.float32) + ee_v[pl.ds(r, 16)][0]
                        for j in range(8):
                            out_v[r, pl.ds(j * 16, 16)] = (
                                B[7][r, pl.ds(j * 16, 16)] * bc)
                    return 0

                lax.fori_loop(0, K // 2, row_body, 0)
                pltpu.sync_copy(out_v, acc.at[B[1]], add=True)
                pltpu.sync_copy(ee_v.at[pl.ds(0, K)], den_sp.at[B[1]], add=True)

            pltpu.sync_copy(src_hbm.at[pl.ds(base, K)], bufs[0][0])
            pltpu.sync_copy(dst_hbm.at[pl.ds(base, K)], bufs[0][1])
            prep_fire_gathers(bufs[0])

            n_pairs = n_chunks // 2

            def pair_body(t, _):
                c0 = t * 2
                fire_sd(bufs[1], c0 + 1)
                wait_gathers(bufs[0])
                wait_sd(bufs[1])
                prep_fire_gathers(bufs[1])
                consume(bufs[0])

                @pl.when(t < n_pairs - 1)
                def _():
                    fire_sd(bufs[0], c0 + 2)

                wait_gathers(bufs[1])

                @pl.when(t < n_pairs - 1)
                def _():
                    wait_sd(bufs[0])
                    prep_fire_gathers(bufs[0])

                consume(bufs[1])
                return 0

            lax.fori_loop(0, n_pairs, pair_body, 0)
            plsc.subcore_barrier()
            pltpu.sync_copy(acc.at[pl.ds(row0, RPT)],
                            out_hbm.at[pl.ds(g * N + row0, RPT)])

            pltpu.sync_copy(den_sp.at[pl.ds(s * RPT, RPT)], den_stage)
            pltpu.sync_copy(den_stage, den_hbm.at[pl.ds(g * N + s * RPT, RPT)])

            @pl.when(s == N_TILES - 1)
            def _():
                pltpu.sync_copy(acc.at[pl.ds(N - 16, 16)],
                                out_hbm.at[pl.ds(g * N + N - 16, 16)])
                pltpu.sync_copy(den_sp.at[pl.ds(N - 16, 16)],
                                den_stage.at[pl.ds(0, 16)])
                pltpu.sync_copy(den_stage.at[pl.ds(0, 16)],
                                den_hbm.at[pl.ds(g * N + N - 16, 16)])

    return sc_edge


_sc_edge1 = _make_sc_edge(H1, N)
_sc_edge2 = _make_sc_edge(2, 0)


def kernel(feat, edge_index1, edge_index2, W1, a_l1, a_r1, b1, W2, a_l2, a_r2, b2):
    src1, dst1 = edge_index1[0], edge_index1[1]
    src2, dst2 = edge_index2[0], edge_index2[1]

    row1 = jnp.arange(H1 * D1)
    col = jnp.arange(128)
    alf = a_l1.reshape(-1)
    arf = a_r1.reshape(-1)
    A1 = (jnp.where(col[None, :] == (row1 // D1)[:, None], alf[:, None], 0.0)
          + jnp.where(col[None, :] == H1 + (row1 // D1)[:, None], arf[:, None], 0.0))
    A2 = (jnp.where(col[None, :] == 0, a_l2[0][:, None], 0.0)
          + jnp.where(col[None, :] == 1, a_r2[0][:, None], 0.0))
    WA1 = _wa(W1, A1)
    WA2 = _wa(W2, A2)

    hcat1, lr1 = _proj1(feat, W1, WA1)
    el1 = lr1[:, :H1].T.reshape(-1)
    er1 = lr1[:, H1:2 * H1].T.reshape(-1)
    raw1, den1 = _sc_edge1(src1, dst1, hcat1.reshape(H1 * N, 128), el1, er1)
    raw1 = raw1.reshape(H1, N, 128)
    den1 = den1.reshape(H1, N).T

    hcat2, lr2 = _proj2(raw1, den1, b1.reshape(1, -1), W2, WA2)
    el2 = lr2[:, 0]
    er2 = lr2[:, 1]
    raw2, den2 = _sc_edge2(src2, dst2, hcat2.reshape(2 * N, 128), el2, er2)
    raw2 = raw2.reshape(2, N, 128)
    den2 = den2.reshape(2, N).T

    return _final(raw2, den2, b2.reshape(1, -1))


# speedup vs baseline: 45.0141x; 1.3935x over previous
import functools

import jax
import jax.numpy as jnp
from jax import lax
from jax.experimental import pallas as pl
from jax.experimental.pallas import tpu as pltpu
from jax.experimental.pallas import tpu_sc as plsc

N = 10000
E = 320000
D_IN = 128
H1 = 4
D1 = 128
D2 = 256

ROWS = 1000
K = 80
N_TILES = 16
E_PER_TILE = E // N_TILES
RPT = 624
ZROWS = 24


def _wa_body(w_ref, a_ref, o_ref):
    o_ref[...] = jnp.dot(w_ref[...], a_ref[...], preferred_element_type=jnp.float32)


def _wa(W, A):
    return pl.pallas_call(
        _wa_body,
        out_shape=jax.ShapeDtypeStruct((W.shape[0], A.shape[1]), jnp.float32),
    )(W, A)


def _proj1_body(x_ref, w_ref, wa_ref, hcat_ref, lr_ref):
    h = jnp.dot(x_ref[...], w_ref[...], preferred_element_type=jnp.float32)
    for g in range(H1):
        hcat_ref[g] = h[:, g * 128:(g + 1) * 128]
    lr_ref[...] = jnp.dot(x_ref[...], wa_ref[...], preferred_element_type=jnp.float32)


def _proj1(x, W, WA):
    return pl.pallas_call(
        _proj1_body,
        grid=(N // ROWS,),
        in_specs=[
            pl.BlockSpec((ROWS, D_IN), lambda i: (i, 0)),
            pl.BlockSpec((D_IN, H1 * D1), lambda i: (0, 0)),
            pl.BlockSpec((D_IN, 128), lambda i: (0, 0)),
        ],
        out_specs=[
            pl.BlockSpec((H1, ROWS, 128), lambda i: (0, i, 0)),
            pl.BlockSpec((ROWS, 128), lambda i: (i, 0)),
        ],
        out_shape=[
            jax.ShapeDtypeStruct((H1, N, 128), jnp.float32),
            jax.ShapeDtypeStruct((N, 128), jnp.float32),
        ],
    )(x, W, WA)


def _proj2_body(raw_ref, den_ref, b_ref, w_ref, wa_ref, hcat_ref, lr_ref):
    parts = []
    for g in range(H1):
        den = den_ref[:, g:g + 1] + 1e-9
        parts.append(raw_ref[g] / den)
    x1 = jnp.concatenate(parts, axis=1) + b_ref[...]
    x1 = jnp.where(x1 > 0, x1, jnp.exp(jnp.minimum(x1, 0.0)) - 1.0)
    h2 = jnp.dot(x1, w_ref[...], preferred_element_type=jnp.float32)
    hcat_ref[0] = h2[:, :128]
    hcat_ref[1] = h2[:, 128:]
    lr_ref[...] = jnp.dot(x1, wa_ref[...], preferred_element_type=jnp.float32)


def _proj2(raw1, den1, b1, W2, WA2):
    return pl.pallas_call(
        _proj2_body,
        grid=(N // ROWS,),
        in_specs=[
            pl.BlockSpec((H1, ROWS, 128), lambda i: (0, i, 0)),
            pl.BlockSpec((ROWS, H1), lambda i: (i, 0)),
            pl.BlockSpec((1, H1 * D1), lambda i: (0, 0)),
            pl.BlockSpec((H1 * D1, D2), lambda i: (0, 0)),
            pl.BlockSpec((H1 * D1, 128), lambda i: (0, 0)),
        ],
        out_specs=[
            pl.BlockSpec((2, ROWS, 128), lambda i: (0, i, 0)),
            pl.BlockSpec((ROWS, 128), lambda i: (i, 0)),
        ],
        out_shape=[
            jax.ShapeDtypeStruct((2, N, 128), jnp.float32),
            jax.ShapeDtypeStruct((N, 128), jnp.float32),
        ],
    )(raw1, den1, b1, W2, WA2)


def _final_body(raw_ref, den_ref, b_ref, z_ref):
    parts = []
    for g in range(2):
        den = den_ref[:, g:g + 1] + 1e-9
        parts.append(raw_ref[g] / den)
    z_ref[...] = jnp.concatenate(parts, axis=1) + b_ref[...]


def _final(raw2, den2, b2):
    return pl.pallas_call(
        _final_body,
        grid=(N // ROWS,),
        in_specs=[
            pl.BlockSpec((2, ROWS, 128), lambda i: (0, i, 0)),
            pl.BlockSpec((ROWS, 2), lambda i: (i, 0)),
            pl.BlockSpec((1, D2), lambda i: (0, 0)),
        ],
        out_specs=pl.BlockSpec((ROWS, D2), lambda i: (i, 0)),
        out_shape=jax.ShapeDtypeStruct((N, D2), jnp.float32),
    )(raw2, den2, b2)


def _make_sc_edge(G, el_stride):
    g_per_core = G // 2
    n_chunks = E_PER_TILE // K
    mesh = plsc.VectorSubcoreMesh(core_axis_name="c", subcore_axis_name="s")

    @functools.partial(
        pl.kernel,
        mesh=mesh,
        out_type=[
            jax.ShapeDtypeStruct((G * N, 128), jnp.float32),
            jax.ShapeDtypeStruct((G * N,), jnp.float32),
        ],
        scratch_types=[
            pltpu.VMEM((RPT,), jnp.float32),
            pltpu.VMEM((RPT,), jnp.float32),
            pltpu.VMEM((ZROWS, 128), jnp.float32),
            pltpu.VMEM_SHARED((N, 128), jnp.float32),
            pltpu.VMEM_SHARED((N,), jnp.float32),
        ] + 2 * [
            pltpu.VMEM((K,), jnp.int32),
            pltpu.VMEM((K,), jnp.int32),
            pltpu.VMEM((K,), jnp.int32),
            pltpu.VMEM((K,), jnp.int32),
            pltpu.VMEM((K,), jnp.int32),
            pltpu.VMEM((K,), jnp.float32),
            pltpu.VMEM((K,), jnp.float32),
            pltpu.VMEM((K, 128), jnp.float32),
            pltpu.SemaphoreType.DMA,
            pltpu.SemaphoreType.DMA,
            pltpu.SemaphoreType.DMA,
            pltpu.VMEM((K, 128), jnp.float32),
            pltpu.VMEM((K + 16,), jnp.float32),
            pltpu.VMEM((K,), jnp.int32),
            pltpu.SemaphoreType.DMA,
        ],
    )
    def sc_edge(src_hbm, dst_hbm, hcat_hbm, el_hbm, er_hbm, out_hbm, den_hbm,
                zbuf1, den_stage, zbuf, acc, den_sp, *bufflat):
        bufs = (bufflat[:15], bufflat[15:])
        c = lax.axis_index("c")
        s = lax.axis_index("s")
        zero16 = jnp.zeros((16,), jnp.float32)

        def zero_zbuf(r, _):
            for j in range(8):
                zbuf[r, pl.ds(j * 16, 16)] = zero16
            return 0

        lax.fori_loop(0, ZROWS, zero_zbuf, 0)

        def zero_zbuf1(i, _):
            zbuf1[pl.ds(i * 16, 16)] = zero16
            return 0

        lax.fori_loop(0, RPT // 16, zero_zbuf1, 0)

        base = s * E_PER_TILE
        row0 = s * RPT

        for kg in range(g_per_core):
            g = c * g_per_core + kg
            pltpu.sync_copy(zbuf1, den_sp.at[pl.ds(s * RPT, RPT)])
            for z in range(RPT // ZROWS):
                pltpu.sync_copy(zbuf, acc.at[pl.ds(row0 + z * ZROWS, ZROWS)])

            @pl.when(s == N_TILES - 1)
            def _():
                pltpu.sync_copy(zbuf.at[pl.ds(0, 16)], acc.at[pl.ds(N - 16, 16)])
                pltpu.sync_copy(zbuf1.at[pl.ds(0, 16)], den_sp.at[pl.ds(N - 16, 16)])

            plsc.subcore_barrier()

            def fire_sd(B, i):
                e0 = base + i * K
                pltpu.async_copy(src_hbm.at[pl.ds(e0, K)], B[0], B[8])
                pltpu.async_copy(dst_hbm.at[pl.ds(e0, K)], B[1], B[8])

            def wait_sd(B):
                pltpu.make_async_copy(src_hbm.at[pl.ds(0, K)], B[0], B[8]).wait()
                pltpu.make_async_copy(dst_hbm.at[pl.ds(0, K)], B[1], B[8]).wait()

            def prep_fire_gathers(B):
                for q in range(K // 16):
                    sl = pl.ds(q * 16, 16)
                    s16 = B[0][sl]
                    d16 = B[1][sl]
                    B[2][sl] = s16 + g * N
                    B[3][sl] = s16 + g * el_stride
                    B[4][sl] = d16 + g * el_stride
                pltpu.async_copy(el_hbm.at[B[3]], B[5], B[9])
                pltpu.async_copy(er_hbm.at[B[4]], B[6], B[9])
                pltpu.async_copy(hcat_hbm.at[B[2]], B[7], B[10])

            def wait_gathers(B):
                pltpu.make_async_copy(el_hbm.at[B[3]], B[5], B[9]).wait()
                pltpu.make_async_copy(er_hbm.at[B[4]], B[6], B[9]).wait()
                pltpu.make_async_copy(hcat_hbm.at[B[2]], B[7], B[10]).wait()

            def consume(B):
                ee_v = B[12]
                out_v = B[11]
                for q in range(K // 16):
                    sl = pl.ds(q * 16, 16)
                    sc = B[5][sl] + B[6][sl]
                    sc = jnp.where(sc >= 0, sc, 0.2 * sc)
                    ee_v[sl] = jnp.exp(sc)
                    B[13][sl] = B[1][sl]

                def row_body(r2, _):
                    for u in range(2):
                        r = r2 * 2 + u
                        bc = jnp.zeros((16,), jnp.float32) + ee_v[pl.ds(r, 16)][0]
                        for j in range(8):
                            out_v[r, pl.ds(j * 16, 16)] = (
                                B[7][r, pl.ds(j * 16, 16)] * bc)
                    return 0

                lax.fori_loop(0, K // 2, row_body, 0)
                pltpu.async_copy(out_v, acc.at[B[13]], B[14], add=True)
                pltpu.async_copy(ee_v.at[pl.ds(0, K)], den_sp.at[B[13]], B[14],
                                 add=True)

            def wait_scatter(B):
                pltpu.make_async_copy(B[11], acc.at[B[13]], B[14]).wait()
                pltpu.make_async_copy(B[12].at[pl.ds(0, K)], den_sp.at[B[13]],
                                      B[14]).wait()

            pltpu.sync_copy(src_hbm.at[pl.ds(base, K)], bufs[0][0])
            pltpu.sync_copy(dst_hbm.at[pl.ds(base, K)], bufs[0][1])
            prep_fire_gathers(bufs[0])

            n_pairs = n_chunks // 2

            def pair_body(t, _):
                c0 = t * 2
                fire_sd(bufs[1], c0 + 1)
                wait_gathers(bufs[0])
                wait_sd(bufs[1])
                prep_fire_gathers(bufs[1])

                @pl.when(t > 0)
                def _():
                    wait_scatter(bufs[0])

                consume(bufs[0])

                @pl.when(t < n_pairs - 1)
                def _():
                    fire_sd(bufs[0], c0 + 2)

                wait_gathers(bufs[1])

                @pl.when(t < n_pairs - 1)
                def _():
                    wait_sd(bufs[0])
                    prep_fire_gathers(bufs[0])

                @pl.when(t > 0)
                def _():
                    wait_scatter(bufs[1])

                consume(bufs[1])
                return 0

            lax.fori_loop(0, n_pairs, pair_body, 0)
            wait_scatter(bufs[0])
            wait_scatter(bufs[1])
            plsc.subcore_barrier()
            pltpu.sync_copy(acc.at[pl.ds(row0, RPT)],
                            out_hbm.at[pl.ds(g * N + row0, RPT)])

            pltpu.sync_copy(den_sp.at[pl.ds(s * RPT, RPT)], den_stage)
            pltpu.sync_copy(den_stage, den_hbm.at[pl.ds(g * N + s * RPT, RPT)])

            @pl.when(s == N_TILES - 1)
            def _():
                pltpu.sync_copy(acc.at[pl.ds(N - 16, 16)],
                                out_hbm.at[pl.ds(g * N + N - 16, 16)])
                pltpu.sync_copy(den_sp.at[pl.ds(N - 16, 16)],
                                den_stage.at[pl.ds(0, 16)])
                pltpu.sync_copy(den_stage.at[pl.ds(0, 16)],
                                den_hbm.at[pl.ds(g * N + N - 16, 16)])

    return sc_edge


_sc_edge1 = _make_sc_edge(H1, N)
_sc_edge2 = _make_sc_edge(2, 0)


def kernel(feat, edge_index1, edge_index2, W1, a_l1, a_r1, b1, W2, a_l2, a_r2, b2):
    src1, dst1 = edge_index1[0], edge_index1[1]
    src2, dst2 = edge_index2[0], edge_index2[1]

    row1 = jnp.arange(H1 * D1)
    col = jnp.arange(128)
    alf = a_l1.reshape(-1)
    arf = a_r1.reshape(-1)
    A1 = (jnp.where(col[None, :] == (row1 // D1)[:, None], alf[:, None], 0.0)
          + jnp.where(col[None, :] == H1 + (row1 // D1)[:, None], arf[:, None], 0.0))
    A2 = (jnp.where(col[None, :] == 0, a_l2[0][:, None], 0.0)
          + jnp.where(col[None, :] == 1, a_r2[0][:, None], 0.0))
    WA1 = _wa(W1, A1)
    WA2 = _wa(W2, A2)

    hcat1, lr1 = _proj1(feat, W1, WA1)
    el1 = lr1[:, :H1].T.reshape(-1)
    er1 = lr1[:, H1:2 * H1].T.reshape(-1)
    raw1, den1 = _sc_edge1(src1, dst1, hcat1.reshape(H1 * N, 128), el1, er1)
    raw1 = raw1.reshape(H1, N, 128)
    den1 = den1.reshape(H1, N).T

    hcat2, lr2 = _proj2(raw1, den1, b1.reshape(1, -1), W2, WA2)
    el2 = lr2[:, 0]
    er2 = lr2[:, 1]
    raw2, den2 = _sc_edge2(src2, dst2, hcat2.reshape(2 * N, 128), el2, er2)
    raw2 = raw2.reshape(2, N, 128)
    den2 = den2.reshape(2, N).T

    return _final(raw2, den2, b2.reshape(1, -1))


# speedup vs baseline: 48.8594x; 1.0854x over previous
import functools

import jax
import jax.numpy as jnp
from jax import lax
from jax.experimental import pallas as pl
from jax.experimental.pallas import tpu as pltpu
from jax.experimental.pallas import tpu_sc as plsc

N = 10000
E = 320000
D_IN = 128
H1 = 4
D1 = 128
D2 = 256

ROWS = 1000
K = 80
N_TILES = 16
E_PER_TILE = E // N_TILES
RPT = 624
ZROWS = 24


def _wa_body(w_ref, a_ref, o_ref):
    o_ref[...] = jnp.dot(w_ref[...], a_ref[...], preferred_element_type=jnp.float32)


def _wa(W, A):
    return pl.pallas_call(
        _wa_body,
        out_shape=jax.ShapeDtypeStruct((W.shape[0], A.shape[1]), jnp.float32),
    )(W, A)


def _proj1_body(x_ref, w_ref, wa_ref, hcat_ref, lr_ref):
    h = jnp.dot(x_ref[...], w_ref[...], preferred_element_type=jnp.float32)
    for g in range(H1):
        hcat_ref[g] = h[:, g * 128:(g + 1) * 128]
    lr_ref[...] = jnp.dot(x_ref[...], wa_ref[...], preferred_element_type=jnp.float32)


def _proj1(x, W, WA):
    return pl.pallas_call(
        _proj1_body,
        grid=(N // ROWS,),
        in_specs=[
            pl.BlockSpec((ROWS, D_IN), lambda i: (i, 0)),
            pl.BlockSpec((D_IN, H1 * D1), lambda i: (0, 0)),
            pl.BlockSpec((D_IN, 128), lambda i: (0, 0)),
        ],
        out_specs=[
            pl.BlockSpec((H1, ROWS, 128), lambda i: (0, i, 0)),
            pl.BlockSpec((ROWS, 128), lambda i: (i, 0)),
        ],
        out_shape=[
            jax.ShapeDtypeStruct((H1, N, 128), jnp.float32),
            jax.ShapeDtypeStruct((N, 128), jnp.float32),
        ],
    )(x, W, WA)


def _proj2_body(raw_ref, den_ref, b_ref, w_ref, wa_ref, hcat_ref, lr_ref):
    parts = []
    for g in range(H1):
        den = den_ref[:, g:g + 1] + 1e-9
        parts.append(raw_ref[g] / den)
    x1 = jnp.concatenate(parts, axis=1) + b_ref[...]
    x1 = jnp.where(x1 > 0, x1, jnp.exp(jnp.minimum(x1, 0.0)) - 1.0)
    h2 = jnp.dot(x1, w_ref[...], preferred_element_type=jnp.float32)
    hcat_ref[0] = h2[:, :128]
    hcat_ref[1] = h2[:, 128:]
    lr_ref[...] = jnp.dot(x1, wa_ref[...], preferred_element_type=jnp.float32)


def _proj2(raw1, den1, b1, W2, WA2):
    return pl.pallas_call(
        _proj2_body,
        grid=(N // ROWS,),
        in_specs=[
            pl.BlockSpec((H1, ROWS, 128), lambda i: (0, i, 0)),
            pl.BlockSpec((ROWS, H1), lambda i: (i, 0)),
            pl.BlockSpec((1, H1 * D1), lambda i: (0, 0)),
            pl.BlockSpec((H1 * D1, D2), lambda i: (0, 0)),
            pl.BlockSpec((H1 * D1, 128), lambda i: (0, 0)),
        ],
        out_specs=[
            pl.BlockSpec((2, ROWS, 128), lambda i: (0, i, 0)),
            pl.BlockSpec((ROWS, 128), lambda i: (i, 0)),
        ],
        out_shape=[
            jax.ShapeDtypeStruct((2, N, 128), jnp.float32),
            jax.ShapeDtypeStruct((N, 128), jnp.float32),
        ],
    )(raw1, den1, b1, W2, WA2)


def _final_body(raw_ref, den_ref, b_ref, z_ref):
    parts = []
    for g in range(2):
        den = den_ref[:, g:g + 1] + 1e-9
        parts.append(raw_ref[g] / den)
    z_ref[...] = jnp.concatenate(parts, axis=1) + b_ref[...]


def _final(raw2, den2, b2):
    return pl.pallas_call(
        _final_body,
        grid=(N // ROWS,),
        in_specs=[
            pl.BlockSpec((2, ROWS, 128), lambda i: (0, i, 0)),
            pl.BlockSpec((ROWS, 2), lambda i: (i, 0)),
            pl.BlockSpec((1, D2), lambda i: (0, 0)),
        ],
        out_specs=pl.BlockSpec((ROWS, D2), lambda i: (i, 0)),
        out_shape=jax.ShapeDtypeStruct((N, D2), jnp.float32),
    )(raw2, den2, b2)


def _make_sc_edge(G, el_stride):
    g_per_core = G // 2
    n_chunks = E_PER_TILE // K
    mesh = plsc.VectorSubcoreMesh(core_axis_name="c", subcore_axis_name="s")

    @functools.partial(
        pl.kernel,
        mesh=mesh,
        out_type=[
            jax.ShapeDtypeStruct((G * N, 128), jnp.float32),
            jax.ShapeDtypeStruct((G * N,), jnp.float32),
        ],
        scratch_types=[
            pltpu.VMEM((RPT,), jnp.float32),
            pltpu.VMEM((RPT,), jnp.float32),
            pltpu.VMEM((ZROWS, 128), jnp.float32),
            pltpu.VMEM_SHARED((N, 128), jnp.float32),
            pltpu.VMEM_SHARED((N,), jnp.float32),
        ] + 2 * [
            pltpu.VMEM((K,), jnp.int32),
            pltpu.VMEM((K,), jnp.int32),
            pltpu.VMEM((K,), jnp.int32),
            pltpu.VMEM((K,), jnp.int32),
            pltpu.VMEM((K,), jnp.int32),
            pltpu.VMEM((K,), jnp.float32),
            pltpu.VMEM((K,), jnp.float32),
            pltpu.VMEM((K, 128), jnp.float32),
            pltpu.SemaphoreType.DMA,
            pltpu.SemaphoreType.DMA,
            pltpu.SemaphoreType.DMA,
            pltpu.VMEM((K, 128), jnp.float32),
            pltpu.VMEM((K + 16,), jnp.float32),
            pltpu.VMEM((K,), jnp.int32),
            pltpu.SemaphoreType.DMA,
        ],
    )
    def sc_edge(src_hbm, dst_hbm, hcat_hbm, el_hbm, er_hbm, out_hbm, den_hbm,
                zbuf1, den_stage, zbuf, acc, den_sp, *bufflat):
        bufs = (bufflat[:15], bufflat[15:])
        c = lax.axis_index("c")
        s = lax.axis_index("s")
        zero16 = jnp.zeros((16,), jnp.float32)

        def zero_zbuf(r, _):
            for j in range(8):
                zbuf[r, pl.ds(j * 16, 16)] = zero16
            return 0

        lax.fori_loop(0, ZROWS, zero_zbuf, 0)

        def zero_zbuf1(i, _):
            zbuf1[pl.ds(i * 16, 16)] = zero16
            return 0

        lax.fori_loop(0, RPT // 16, zero_zbuf1, 0)

        base = s * E_PER_TILE
        row0 = s * RPT

        for kg in range(g_per_core):
            g = c * g_per_core + kg
            pltpu.sync_copy(zbuf1, den_sp.at[pl.ds(s * RPT, RPT)])
            for z in range(RPT // ZROWS):
                pltpu.sync_copy(zbuf, acc.at[pl.ds(row0 + z * ZROWS, ZROWS)])

            @pl.when(s == N_TILES - 1)
            def _():
                pltpu.sync_copy(zbuf.at[pl.ds(0, 16)], acc.at[pl.ds(N - 16, 16)])
                pltpu.sync_copy(zbuf1.at[pl.ds(0, 16)], den_sp.at[pl.ds(N - 16, 16)])

            plsc.subcore_barrier()

            def fire_sd(B, i):
                e0 = base + i * K
                pltpu.async_copy(src_hbm.at[pl.ds(e0, K)], B[0], B[8])
                pltpu.async_copy(dst_hbm.at[pl.ds(e0, K)], B[1], B[8])

            def wait_sd(B):
                pltpu.make_async_copy(src_hbm.at[pl.ds(0, K)], B[0], B[8]).wait()
                pltpu.make_async_copy(dst_hbm.at[pl.ds(0, K)], B[1], B[8]).wait()

            def prep_fire_gathers(B):
                for q in range(K // 16):
                    sl = pl.ds(q * 16, 16)
                    s16 = B[0][sl]
                    d16 = B[1][sl]
                    B[2][sl] = s16 + g * N
                    B[3][sl] = s16 + g * el_stride
                    B[4][sl] = d16 + g * el_stride
                pltpu.async_copy(el_hbm.at[B[3]], B[5], B[9])
                pltpu.async_copy(er_hbm.at[B[4]], B[6], B[9])
                pltpu.async_copy(hcat_hbm.at[B[2]], B[7], B[10])

            def wait_gathers(B):
                pltpu.make_async_copy(el_hbm.at[B[3]], B[5], B[9]).wait()
                pltpu.make_async_copy(er_hbm.at[B[4]], B[6], B[9]).wait()
                pltpu.make_async_copy(hcat_hbm.at[B[2]], B[7], B[10]).wait()

            def consume(B):
                ee_v = B[12]
                out_v = B[11]
                for q in range(K // 16):
                    sl = pl.ds(q * 16, 16)
                    sc = B[5][sl] + B[6][sl]
                    sc = jnp.where(sc >= 0, sc, 0.2 * sc)
                    ee_v[sl] = jnp.exp(sc)
                    B[13][sl] = B[1][sl]

                def row_body(r4, _):
                    bcs = []
                    for u in range(4):
                        r = r4 * 4 + u
                        bcs.append(
                            jnp.zeros((16,), jnp.float32) + ee_v[pl.ds(r, 16)][0])
                    for u in range(4):
                        r = r4 * 4 + u
                        for j in range(8):
                            out_v[r, pl.ds(j * 16, 16)] = (
                                B[7][r, pl.ds(j * 16, 16)] * bcs[u])
                    return 0

                lax.fori_loop(0, K // 4, row_body, 0)
                pltpu.async_copy(out_v, acc.at[B[13]], B[14], add=True)
                pltpu.async_copy(ee_v.at[pl.ds(0, K)], den_sp.at[B[13]], B[14],
                                 add=True)

            def wait_scatter(B):
                pltpu.make_async_copy(B[11], acc.at[B[13]], B[14]).wait()
                pltpu.make_async_copy(B[12].at[pl.ds(0, K)], den_sp.at[B[13]],
                                      B[14]).wait()

            pltpu.sync_copy(src_hbm.at[pl.ds(base, K)], bufs[0][0])
            pltpu.sync_copy(dst_hbm.at[pl.ds(base, K)], bufs[0][1])
            prep_fire_gathers(bufs[0])

            n_pairs = n_chunks // 2

            def pair_body(t, _):
                c0 = t * 2
                fire_sd(bufs[1], c0 + 1)
                wait_gathers(bufs[0])
                wait_sd(bufs[1])
                prep_fire_gathers(bufs[1])

                @pl.when(t > 0)
                def _():
                    wait_scatter(bufs[0])

                consume(bufs[0])

                @pl.when(t < n_pairs - 1)
                def _():
                    fire_sd(bufs[0], c0 + 2)

                wait_gathers(bufs[1])

                @pl.when(t < n_pairs - 1)
                def _():
                    wait_sd(bufs[0])
                    prep_fire_gathers(bufs[0])

                @pl.when(t > 0)
                def _():
                    wait_scatter(bufs[1])

                consume(bufs[1])
                return 0

            lax.fori_loop(0, n_pairs, pair_body, 0)
            wait_scatter(bufs[0])
            wait_scatter(bufs[1])
            plsc.subcore_barrier()
            pltpu.sync_copy(acc.at[pl.ds(row0, RPT)],
                            out_hbm.at[pl.ds(g * N + row0, RPT)])

            pltpu.sync_copy(den_sp.at[pl.ds(s * RPT, RPT)], den_stage)
            pltpu.sync_copy(den_stage, den_hbm.at[pl.ds(g * N + s * RPT, RPT)])

            @pl.when(s == N_TILES - 1)
            def _():
                pltpu.sync_copy(acc.at[pl.ds(N - 16, 16)],
                                out_hbm.at[pl.ds(g * N + N - 16, 16)])
                pltpu.sync_copy(den_sp.at[pl.ds(N - 16, 16)],
                                den_stage.at[pl.ds(0, 16)])
                pltpu.sync_copy(den_stage.at[pl.ds(0, 16)],
                                den_hbm.at[pl.ds(g * N + N - 16, 16)])

    return sc_edge


_sc_edge1 = _make_sc_edge(H1, N)
_sc_edge2 = _make_sc_edge(2, 0)


def kernel(feat, edge_index1, edge_index2, W1, a_l1, a_r1, b1, W2, a_l2, a_r2, b2):
    src1, dst1 = edge_index1[0], edge_index1[1]
    src2, dst2 = edge_index2[0], edge_index2[1]

    row1 = jnp.arange(H1 * D1)
    col = jnp.arange(128)
    alf = a_l1.reshape(-1)
    arf = a_r1.reshape(-1)
    A1 = (jnp.where(col[None, :] == (row1 // D1)[:, None], alf[:, None], 0.0)
          + jnp.where(col[None, :] == H1 + (row1 // D1)[:, None], arf[:, None], 0.0))
    A2 = (jnp.where(col[None, :] == 0, a_l2[0][:, None], 0.0)
          + jnp.where(col[None, :] == 1, a_r2[0][:, None], 0.0))
    WA1 = _wa(W1, A1)
    WA2 = _wa(W2, A2)

    hcat1, lr1 = _proj1(feat, W1, WA1)
    el1 = lr1[:, :H1].T.reshape(-1)
    er1 = lr1[:, H1:2 * H1].T.reshape(-1)
    raw1, den1 = _sc_edge1(src1, dst1, hcat1.reshape(H1 * N, 128), el1, er1)
    raw1 = raw1.reshape(H1, N, 128)
    den1 = den1.reshape(H1, N).T

    hcat2, lr2 = _proj2(raw1, den1, b1.reshape(1, -1), W2, WA2)
    el2 = lr2[:, 0]
    er2 = lr2[:, 1]
    raw2, den2 = _sc_edge2(src2, dst2, hcat2.reshape(2 * N, 128), el2, er2)
    raw2 = raw2.reshape(2, N, 128)
    den2 = den2.reshape(2, N).T

    return _final(raw2, den2, b2.reshape(1, -1))
